# baseline (speedup 1.0000x reference)
import functools

import jax
import jax.numpy as jnp
from jax import lax
from jax.experimental import pallas as pl
from jax.experimental.pallas import tpu as pltpu
from jax.experimental.pallas import tpu_sc as plsc


DIM = 64
CHUNK = 128
NROW = 4
NTR = 2

_DNUMS = lax.GatherDimensionNumbers(
    offset_dims=(), collapsed_slice_dims=(0,), start_index_map=(0,)
)


def _perm(v, idx2d):
  return lax.gather(v, idx2d, _DNUMS, slice_sizes=(1,),
                    mode=lax.GatherScatterMode.PROMISE_IN_BOUNDS)


def _butterfly(load, store):
  lane = lax.iota(jnp.int32, 16)
  vs = [load(i) for i in range(16)]
  for s in (1, 2, 4, 8):
    idx2d = (lane ^ s).reshape(16, 1)
    mask = (lane & s) == 0
    nv = list(vs)
    for i in range(16):
      if i & s == 0:
        j = i | s
        pa = _perm(vs[j], idx2d)
        pb = _perm(vs[i], idx2d)
        nv[i] = jnp.where(mask, vs[i], pa)
        nv[j] = jnp.where(mask, pb, vs[j])
    vs = nv
  for j in range(16):
    store(j, vs[j])


def _make_detranspose(vocab: int):
  mesh = plsc.VectorSubcoreMesh(core_axis_name="c", subcore_axis_name="s")
  n_slabs = vocab // CHUNK
  tail = vocab - n_slabs * CHUNK
  base = n_slabs // 32
  extra = n_slabs - base * 32

  @functools.partial(
      pl.kernel,
      out_type=jax.ShapeDtypeStruct((vocab // 2, 2 * DIM), jnp.float32),
      mesh=mesh,
      scratch_types=[
          pltpu.VMEM((4, DIM, CHUNK), jnp.float32),
          pltpu.VMEM((2, DIM, CHUNK), jnp.float32),
          pltpu.SemaphoreType.DMA((4,)),
          pltpu.SemaphoreType.DMA((2,)),
      ],
      compiler_params=pltpu.CompilerParams(
          use_tc_tiling_on_sc=True, needs_layout_passes=False
      ),
  )
  def detrans_kernel(tbl_hbm, out_hbm, src_v, dst_v, gsem, ssem):
    num_cores = lax.axis_size("c")
    wid = lax.axis_index("s") * num_cores + lax.axis_index("c")
    start = wid * base

    def src_slab(it):
      return tbl_hbm.at[:, pl.ds(pl.multiple_of(it * CHUNK, CHUNK), CHUNK)]

    def out_slab(it):
      return out_hbm.at[pl.ds(it * DIM, DIM), :]

    def transpose_slab(b, tb, nblk=CHUNK // 16):
      @pl.loop(0, (DIM // 16) * nblk)
      def _(blk):
        db16 = (blk // nblk) * 16
        ig16 = (blk % nblk) * 16

        def ld(i):
          return src_v[b, db16 + i, pl.ds(ig16, 16)]

        def st(j, vreg):
          i = ig16 + j
          dst_v[tb, i // 2, pl.ds((i % 2) * DIM + db16, 16)] = vreg

        _butterfly(ld, st)

    transpose_slab_src = transpose_slab

    for j in range(2):
      pltpu.async_copy(src_slab(start + j), src_v.at[j], gsem.at[j])

    @pl.loop(0, base, step=4)
    def _(j0):
      for u in range(4):
        k = j0 + u
        it = start + k
        b = u % 4
        tb = u % 2
        pltpu.make_async_copy(src_slab(it), src_v.at[b], gsem.at[b]).wait()

        @pl.when(k + 2 < base)
        def _():
          pltpu.async_copy(
              src_slab(it + 2), src_v.at[(u + 2) % 4], gsem.at[(u + 2) % 4]
          )

        @pl.when(k >= 2)
        def _():
          pltpu.make_async_copy(
              dst_v.at[tb], out_slab(it - 2), ssem.at[tb]
          ).wait()

        transpose_slab_src(b, tb)
        pltpu.async_copy(dst_v.at[tb], out_slab(it), ssem.at[tb])

    for t in range(2):
      k = base - 2 + t
      pltpu.make_async_copy(
          dst_v.at[k % 2], out_slab(start + k), ssem.at[k % 2]
      ).wait()

    @pl.when(wid < extra)
    def _():
      it = n_slabs - extra + wid
      pltpu.sync_copy(src_slab(it), src_v.at[0])
      transpose_slab(0, 0)
      pltpu.sync_copy(dst_v.at[0], out_slab(it))

    if tail:
      @pl.when(wid == extra)
      def _():
        for d in range(DIM):
          pltpu.async_copy(
              tbl_hbm.at[d, pl.ds(n_slabs * CHUNK, tail)],
              src_v.at[0, d, pl.ds(0, tail)],
              gsem.at[0],
          )
        for d in range(DIM):
          pltpu.make_async_copy(
              tbl_hbm.at[d, pl.ds(n_slabs * CHUNK, tail)],
              src_v.at[0, d, pl.ds(0, tail)],
              gsem.at[0],
          ).wait()
        transpose_slab(0, 0, tail // 16)
        pltpu.sync_copy(
            dst_v.at[0, pl.ds(0, tail // 2), :],
            out_hbm.at[pl.ds(n_slabs * DIM, tail // 2), :],
        )

  return detrans_kernel


def _make_gather(num_workers: int, seq: int, btiles: int, vocab: int):
  mesh = plsc.VectorSubcoreMesh(core_axis_name="c", subcore_axis_name="s")
  n_chunks = seq * btiles
  cpw = n_chunks // num_workers
  assert cpw * num_workers == n_chunks
  assert cpw % NROW == 0

  @functools.partial(
      pl.kernel,
      out_type=jax.ShapeDtypeStruct((seq, 8, btiles, 8, CHUNK), jnp.float32),
      mesh=mesh,
      scratch_types=[
          pltpu.VMEM((cpw, CHUNK), jnp.int32),
          pltpu.VMEM((NROW, CHUNK, DIM), jnp.float32),
          pltpu.VMEM((NTR, 8, 8, CHUNK), jnp.float32),
          pltpu.SemaphoreType.DMA((NROW,)),
          pltpu.SemaphoreType.DMA((NTR,)),
      ],
      compiler_params=pltpu.CompilerParams(
          use_tc_tiling_on_sc=False, needs_layout_passes=False
      ),
  )
  def gather_kernel(ids_hbm, table_hbm, out_hbm, idx_v, rows_v, tr2_v,
                    gsem, ssem):
    num_cores = lax.axis_size("c")
    wid = lax.axis_index("s") * num_cores + lax.axis_index("c")
    c0 = wid * cpw

    def out_block(c):
      return out_hbm.at[c // btiles, :, c % btiles]

    pltpu.sync_copy(ids_hbm.at[pl.ds(c0, cpw)], idx_v)

    for j in range(3):
      pltpu.async_copy(table_hbm.at[idx_v.at[j]], rows_v.at[j], gsem.at[j])

    @pl.loop(0, cpw, step=NROW)
    def _(j0):
      for u in range(NROW):
        cj = j0 + u
        b = u % NROW
        tb = u % NTR

        pltpu.make_async_copy(
            table_hbm.at[idx_v.at[cj]], rows_v.at[b], gsem.at[b]
        ).wait()

        @pl.when(cj + 3 < cpw)
        def _():
          pltpu.async_copy(
              table_hbm.at[idx_v.at[cj + 3]],
              rows_v.at[(u + 3) % NROW],
              gsem.at[(u + 3) % NROW],
          )

        @pl.when(cj >= NTR)
        def _():
          pltpu.make_async_copy(
              tr2_v.at[tb], out_block(c0 + cj - NTR), ssem.at[tb]
          ).wait()

        @pl.loop(0, 32)
        def _(blk):
          g16 = (blk // 4) * 16
          d16 = (blk % 4) * 16

          def ld(i):
            return rows_v[b, g16 + i, pl.ds(d16, 16)]

          def st(jj, vreg):
            d = d16 + jj
            tr2_v[tb, d // 8, d % 8, pl.ds(g16, 16)] = vreg

          _butterfly(ld, st)

        pltpu.async_copy(tr2_v.at[tb], out_block(c0 + cj), ssem.at[tb])

    for t in range(NTR):
      cj = cpw - NTR + t
      pltpu.make_async_copy(
          tr2_v.at[cj % NTR], out_block(c0 + cj), ssem.at[cj % NTR]
      ).wait()

  return gather_kernel


def kernel(input_ids, word_table):
  batch, seq = input_ids.shape
  assert batch % CHUNK == 0
  btiles = batch // CHUNK
  info = plsc.get_sparse_core_info()
  num_workers = info.num_cores * info.num_subcores
  vocab = word_table.shape[0]

  scratch = _make_detranspose(vocab)(word_table.T)
  table_lin = scratch.reshape(vocab, DIM)

  ids_prep = input_ids.T.astype(jnp.int32).reshape(seq * btiles, CHUNK)
  out5d = _make_gather(num_workers, seq, btiles, vocab)(ids_prep, table_lin)
  out = out5d.transpose(2, 4, 0, 1, 3).reshape(batch, seq, DIM)
  return out


# speedup vs baseline: 1.0032x; 1.0032x over previous
import functools

import jax
import jax.numpy as jnp
from jax import lax
from jax.experimental import pallas as pl
from jax.experimental.pallas import tpu as pltpu
from jax.experimental.pallas import tpu_sc as plsc


DIM = 64
CHUNK = 128
NROW = 4
NTR = 2

_DNUMS = lax.GatherDimensionNumbers(
    offset_dims=(), collapsed_slice_dims=(0,), start_index_map=(0,)
)


def _perm(v, idx2d):
  return lax.gather(v, idx2d, _DNUMS, slice_sizes=(1,),
                    mode=lax.GatherScatterMode.PROMISE_IN_BOUNDS)


def _butterfly(load, store):
  lane = lax.iota(jnp.int32, 16)
  vs = [load(i) for i in range(16)]
  for s in (1, 2, 4, 8):
    idx2d = (lane ^ s).reshape(16, 1)
    mask = (lane & s) == 0
    nv = list(vs)
    for i in range(16):
      if i & s == 0:
        j = i | s
        pa = _perm(vs[j], idx2d)
        pb = _perm(vs[i], idx2d)
        nv[i] = jnp.where(mask, vs[i], pa)
        nv[j] = jnp.where(mask, pb, vs[j])
    vs = nv
  for j in range(16):
    store(j, vs[j])


def _make_detranspose(vocab: int):
  mesh = plsc.VectorSubcoreMesh(core_axis_name="c", subcore_axis_name="s")
  n_slabs = vocab // CHUNK
  tail = vocab - n_slabs * CHUNK
  base = n_slabs // 32
  extra = n_slabs - base * 32

  @functools.partial(
      pl.kernel,
      out_type=jax.ShapeDtypeStruct((vocab // 2, 2 * DIM), jnp.float32),
      mesh=mesh,
      scratch_types=[
          pltpu.VMEM((4, DIM, CHUNK), jnp.float32),
          pltpu.VMEM((2, DIM, CHUNK), jnp.float32),
          pltpu.SemaphoreType.DMA((4,)),
          pltpu.SemaphoreType.DMA((2,)),
      ],
      compiler_params=pltpu.CompilerParams(
          use_tc_tiling_on_sc=True, needs_layout_passes=False
      ),
  )
  def detrans_kernel(tbl_hbm, out_hbm, src_v, dst_v, gsem, ssem):
    num_cores = lax.axis_size("c")
    wid = lax.axis_index("s") * num_cores + lax.axis_index("c")
    start = wid * base

    def src_slab(it):
      return tbl_hbm.at[:, pl.ds(pl.multiple_of(it * CHUNK, CHUNK), CHUNK)]

    def out_slab(it):
      return out_hbm.at[pl.ds(it * DIM, DIM), :]

    def transpose_slab(b, tb, nblk=CHUNK // 16):
      @pl.loop(0, (DIM // 16) * nblk)
      def _(blk):
        db16 = (blk // nblk) * 16
        ig16 = (blk % nblk) * 16

        def ld(i):
          return src_v[b, db16 + i, pl.ds(ig16, 16)]

        def st(j, vreg):
          i = ig16 + j
          dst_v[tb, i // 2, pl.ds((i % 2) * DIM + db16, 16)] = vreg

        _butterfly(ld, st)

    transpose_slab_src = transpose_slab

    for j in range(2):
      pltpu.async_copy(src_slab(start + j), src_v.at[j], gsem.at[j])

    @pl.loop(0, base, step=4)
    def _(j0):
      for u in range(4):
        k = j0 + u
        it = start + k
        b = u % 4
        tb = u % 2
        pltpu.make_async_copy(src_slab(it), src_v.at[b], gsem.at[b]).wait()

        @pl.when(k + 2 < base)
        def _():
          pltpu.async_copy(
              src_slab(it + 2), src_v.at[(u + 2) % 4], gsem.at[(u + 2) % 4]
          )

        @pl.when(k >= 2)
        def _():
          pltpu.make_async_copy(
              dst_v.at[tb], out_slab(it - 2), ssem.at[tb]
          ).wait()

        transpose_slab_src(b, tb)
        pltpu.async_copy(dst_v.at[tb], out_slab(it), ssem.at[tb])

    for t in range(2):
      k = base - 2 + t
      pltpu.make_async_copy(
          dst_v.at[k % 2], out_slab(start + k), ssem.at[k % 2]
      ).wait()

    @pl.when(wid < extra)
    def _():
      it = n_slabs - extra + wid
      pltpu.sync_copy(src_slab(it), src_v.at[0])
      transpose_slab(0, 0)
      pltpu.sync_copy(dst_v.at[0], out_slab(it))

    if tail:
      @pl.when(wid == extra)
      def _():
        for d in range(DIM):
          pltpu.async_copy(
              tbl_hbm.at[d, pl.ds(n_slabs * CHUNK, tail)],
              src_v.at[0, d, pl.ds(0, tail)],
              gsem.at[0],
          )
        for d in range(DIM):
          pltpu.make_async_copy(
              tbl_hbm.at[d, pl.ds(n_slabs * CHUNK, tail)],
              src_v.at[0, d, pl.ds(0, tail)],
              gsem.at[0],
          ).wait()
        transpose_slab(0, 0, tail // 16)
        pltpu.sync_copy(
            dst_v.at[0, pl.ds(0, tail // 2), :],
            out_hbm.at[pl.ds(n_slabs * DIM, tail // 2), :],
        )

  return detrans_kernel


def _make_gather(num_workers: int, seq: int, btiles: int, vocab: int):
  mesh = plsc.VectorSubcoreMesh(core_axis_name="c", subcore_axis_name="s")
  n_chunks = seq * btiles
  cpw = n_chunks // num_workers
  assert cpw * num_workers == n_chunks
  assert cpw % NROW == 0

  @functools.partial(
      pl.kernel,
      out_type=jax.ShapeDtypeStruct((seq, 8, btiles, 8, CHUNK), jnp.float32),
      mesh=mesh,
      scratch_types=[
          pltpu.VMEM((cpw, CHUNK), jnp.int32),
          pltpu.VMEM((NROW, CHUNK, DIM), jnp.float32),
          pltpu.VMEM((NTR, 8, 8, CHUNK), jnp.float32),
          pltpu.SemaphoreType.DMA((NROW,)),
          pltpu.SemaphoreType.DMA((NTR,)),
      ],
      compiler_params=pltpu.CompilerParams(
          use_tc_tiling_on_sc=False, needs_layout_passes=False
      ),
  )
  def gather_kernel(ids_hbm, table_hbm, out_hbm, idx_v, rows_v, tr2_v,
                    gsem, ssem):
    num_cores = lax.axis_size("c")
    wid = lax.axis_index("s") * num_cores + lax.axis_index("c")
    c0 = wid * cpw

    def out_block(c):
      return out_hbm.at[c // btiles, :, c % btiles]

    pltpu.sync_copy(ids_hbm.at[pl.ds(c0, cpw)], idx_v)

    for j in range(2):
      pltpu.async_copy(table_hbm.at[idx_v.at[j]], rows_v.at[j], gsem.at[j])

    @pl.loop(0, cpw, step=NROW)
    def _(j0):
      for u in range(NROW):
        cj = j0 + u
        b = u % NROW
        tb = u % NTR

        pltpu.make_async_copy(
            table_hbm.at[idx_v.at[cj]], rows_v.at[b], gsem.at[b]
        ).wait()

        @pl.when(cj + 2 < cpw)
        def _():
          pltpu.async_copy(
              table_hbm.at[idx_v.at[cj + 2]],
              rows_v.at[(u + 2) % NROW],
              gsem.at[(u + 2) % NROW],
          )

        @pl.when(cj >= NTR)
        def _():
          pltpu.make_async_copy(
              tr2_v.at[tb], out_block(c0 + cj - NTR), ssem.at[tb]
          ).wait()

        @pl.loop(0, 32)
        def _(blk):
          g16 = (blk // 4) * 16
          d16 = (blk % 4) * 16

          def ld(i):
            return rows_v[b, g16 + i, pl.ds(d16, 16)]

          def st(jj, vreg):
            d = d16 + jj
            tr2_v[tb, d // 8, d % 8, pl.ds(g16, 16)] = vreg

          _butterfly(ld, st)

        pltpu.async_copy(tr2_v.at[tb], out_block(c0 + cj), ssem.at[tb])

    for t in range(NTR):
      cj = cpw - NTR + t
      pltpu.make_async_copy(
          tr2_v.at[cj % NTR], out_block(c0 + cj), ssem.at[cj % NTR]
      ).wait()

  return gather_kernel


def kernel(input_ids, word_table):
  batch, seq = input_ids.shape
  assert batch % CHUNK == 0
  btiles = batch // CHUNK
  info = plsc.get_sparse_core_info()
  num_workers = info.num_cores * info.num_subcores
  vocab = word_table.shape[0]

  scratch = _make_detranspose(vocab)(word_table.T)
  table_lin = scratch.reshape(vocab, DIM)

  ids_prep = input_ids.T.astype(jnp.int32).reshape(seq * btiles, CHUNK)
  out5d = _make_gather(num_workers, seq, btiles, vocab)(ids_prep, table_lin)
  out = out5d.transpose(2, 4, 0, 1, 3).reshape(batch, seq, DIM)
  return out
